# Initial kernel scaffold; baseline (speedup 1.0000x reference)
import functools

import jax
import jax.numpy as jnp
from jax import lax
from jax.experimental import pallas as pl
from jax.experimental.pallas import tpu as pltpu
from jax.experimental.pallas import tpu_sc as plsc

N = 10000
E = 320000
D = 128
H = 128
O = 128

NC = 2
NS = 16
NW = NC * NS
K = 128
PB = -(-E // (NW * K))
EP = NW * PB * K
NP = 10240
RPT = N // NS
ZR = 125

BLK = 2000
GRID = N // BLK

_f32 = jnp.float32


def _tc_in_body(x_ref, w_ref, asrc_ref, adst_ref, h_ref, es_ref, ed_ref):
    h = jnp.dot(x_ref[...], w_ref[...], preferred_element_type=_f32)
    h_ref[...] = h
    es_ref[...] = jnp.dot(h, asrc_ref[...], preferred_element_type=_f32)
    ed_ref[...] = jnp.dot(h, adst_ref[...], preferred_element_type=_f32)


def _tc_in(x, W, a_src, a_dst):
    return pl.pallas_call(
        _tc_in_body,
        grid=(GRID,),
        in_specs=[
            pl.BlockSpec((BLK, D), lambda i: (i, 0)),
            pl.BlockSpec((D, H), lambda i: (0, 0)),
            pl.BlockSpec((H, 1), lambda i: (0, 0)),
            pl.BlockSpec((H, 1), lambda i: (0, 0)),
        ],
        out_specs=[
            pl.BlockSpec((BLK, H), lambda i: (i, 0)),
            pl.BlockSpec((BLK, 1), lambda i: (i, 0)),
            pl.BlockSpec((BLK, 1), lambda i: (i, 0)),
        ],
        out_shape=[
            jax.ShapeDtypeStruct((N, H), _f32),
            jax.ShapeDtypeStruct((N, 1), _f32),
            jax.ShapeDtypeStruct((N, 1), _f32),
        ],
    )(x, W, a_src.reshape(H, 1), a_dst.reshape(H, 1))


def _combine(acc_ref, den_ref, h_ref, es_ref, ed_ref, b_ref):
    t = es_ref[...] + ed_ref[...]
    sw = jnp.exp(jnp.maximum(t, 0.2 * t))
    num = acc_ref[0] + acc_ref[1] + sw * h_ref[...]
    den = den_ref[0] + den_ref[1] + sw
    return num / den + b_ref[...]


def _tc_mid_body(acc_ref, den_ref, h_ref, es_ref, ed_ref, b_ref, w_ref,
                 asrc_ref, adst_ref, h2_ref, es2_ref, ed2_ref):
    g = jnp.maximum(_combine(acc_ref, den_ref, h_ref, es_ref, ed_ref, b_ref), 0.0)
    h2 = jnp.dot(g, w_ref[...], preferred_element_type=_f32)
    h2_ref[...] = h2
    es2_ref[...] = jnp.dot(h2, asrc_ref[...], preferred_element_type=_f32)
    ed2_ref[...] = jnp.dot(h2, adst_ref[...], preferred_element_type=_f32)


def _tc_mid(acc, den, h, es, ed, b, W, a_src, a_dst):
    return pl.pallas_call(
        _tc_mid_body,
        grid=(GRID,),
        in_specs=[
            pl.BlockSpec((2, BLK, H), lambda i: (0, i, 0)),
            pl.BlockSpec((2, BLK, 1), lambda i: (0, i, 0)),
            pl.BlockSpec((BLK, H), lambda i: (i, 0)),
            pl.BlockSpec((BLK, 1), lambda i: (i, 0)),
            pl.BlockSpec((BLK, 1), lambda i: (i, 0)),
            pl.BlockSpec((1, H), lambda i: (0, 0)),
            pl.BlockSpec((H, H), lambda i: (0, 0)),
            pl.BlockSpec((H, 1), lambda i: (0, 0)),
            pl.BlockSpec((H, 1), lambda i: (0, 0)),
        ],
        out_specs=[
            pl.BlockSpec((BLK, H), lambda i: (i, 0)),
            pl.BlockSpec((BLK, 1), lambda i: (i, 0)),
            pl.BlockSpec((BLK, 1), lambda i: (i, 0)),
        ],
        out_shape=[
            jax.ShapeDtypeStruct((N, H), _f32),
            jax.ShapeDtypeStruct((N, 1), _f32),
            jax.ShapeDtypeStruct((N, 1), _f32),
        ],
    )(acc, den, h, es, ed, b.reshape(1, H), W,
      a_src.reshape(H, 1), a_dst.reshape(H, 1))


def _tc_out_body(acc_ref, den_ref, h_ref, es_ref, ed_ref, b_ref, w_ref,
                 bout_ref, out_ref):
    g = jnp.maximum(_combine(acc_ref, den_ref, h_ref, es_ref, ed_ref, b_ref), 0.0)
    out_ref[...] = (jnp.dot(g, w_ref[...], preferred_element_type=_f32)
                    + bout_ref[...])


def _tc_out(acc, den, h, es, ed, b, Wout, bout):
    return pl.pallas_call(
        _tc_out_body,
        grid=(GRID,),
        in_specs=[
            pl.BlockSpec((2, BLK, H), lambda i: (0, i, 0)),
            pl.BlockSpec((2, BLK, 1), lambda i: (0, i, 0)),
            pl.BlockSpec((BLK, H), lambda i: (i, 0)),
            pl.BlockSpec((BLK, 1), lambda i: (i, 0)),
            pl.BlockSpec((BLK, 1), lambda i: (i, 0)),
            pl.BlockSpec((1, H), lambda i: (0, 0)),
            pl.BlockSpec((H, O), lambda i: (0, 0)),
            pl.BlockSpec((1, O), lambda i: (0, 0)),
        ],
        out_specs=pl.BlockSpec((BLK, O), lambda i: (i, 0)),
        out_shape=jax.ShapeDtypeStruct((N, O), _f32),
    )(acc, den, h, es, ed, b.reshape(1, H), Wout, bout.reshape(1, O))


def _sc_edge_body(h_hbm, es_hbm, ed_hbm, src_hbm, dst_hbm,
                  acc_out, den_out,
                  es_v, ed_v, src_v, dst_v, rows_v, w_v, stage_v,
                  acc_sh, den_sh, sem):
    c = lax.axis_index("c")
    s = lax.axis_index("s")
    wid = c * NS + s
    base = wid * PB

    pltpu.sync_copy(es_hbm, es_v)
    pltpu.sync_copy(ed_hbm, ed_v)
    pltpu.sync_copy(src_hbm.at[pl.ds(base, PB)], src_v)
    pltpu.sync_copy(dst_hbm.at[pl.ds(base, PB)], dst_v)

    zero16 = jnp.zeros((16,), _f32)

    @pl.loop(0, ZR)
    def _zero_stage(i):
        for j in range(H // 16):
            stage_v[i, pl.ds(j * 16, 16)] = zero16

    for k in range(RPT // ZR):
        pltpu.sync_copy(stage_v, acc_sh.at[pl.ds(s * RPT + k * ZR, ZR)])
    for k in range(5):
        pltpu.sync_copy(stage_v.at[0].at[pl.ds(0, K)],
                        den_sh.at[pl.ds(s * 640 + k * K, K)])

    plsc.subcore_barrier()

    lane = lax.iota(jnp.int32, 16)

    @pl.loop(0, PB)
    def _batch(b):
        pltpu.sync_copy(h_hbm.at[src_v.at[b]], rows_v)

        for j in range(K // 16):
            sidx = src_v[b, pl.ds(j * 16, 16)]
            didx = dst_v[b, pl.ds(j * 16, 16)]
            t = plsc.load_gather(es_v, [sidx]) + plsc.load_gather(ed_v, [didx])
            w = jnp.exp(jnp.maximum(t, 0.2 * t))
            gid = (base + b) * K + j * 16 + lane
            w_v[pl.ds(j * 16, 16)] = jnp.where(gid < E, w, 0.0)

        @pl.loop(0, K)
        def _row(r):
            wr = plsc.load_gather(w_v, [jnp.full((16,), 0, jnp.int32) + r])
            for j in range(H // 16):
                rows_v[r, pl.ds(j * 16, 16)] = rows_v[r, pl.ds(j * 16, 16)] * wr

        pltpu.sync_copy(rows_v, acc_sh.at[dst_v.at[b]], add=True)
        pltpu.sync_copy(w_v, den_sh.at[dst_v.at[b]], add=True)

    plsc.subcore_barrier()

    for k in range(RPT // ZR):
        off = s * RPT + k * ZR
        pltpu.sync_copy(acc_sh.at[pl.ds(off, ZR)], stage_v)
        pltpu.sync_copy(stage_v, acc_out.at[c].at[pl.ds(off, ZR)])
    for k in range(5):
        off = s * 640 + k * K
        pltpu.sync_copy(den_sh.at[pl.ds(off, K)], w_v)
        pltpu.sync_copy(w_v, den_out.at[c].at[pl.ds(off, K)])


_sc_edge = pl.kernel(
    _sc_edge_body,
    out_type=[
        jax.ShapeDtypeStruct((NC, N, H), _f32),
        jax.ShapeDtypeStruct((NC, NP), _f32),
    ],
    mesh=plsc.VectorSubcoreMesh(core_axis_name="c", subcore_axis_name="s",
                                num_cores=NC, num_subcores=NS),
    scratch_types=[
        pltpu.VMEM((N,), _f32),
        pltpu.VMEM((N,), _f32),
        pltpu.VMEM((PB, K), jnp.int32),
        pltpu.VMEM((PB, K), jnp.int32),
        pltpu.VMEM((K, H), _f32),
        pltpu.VMEM((K,), _f32),
        pltpu.VMEM((ZR, H), _f32),
        pltpu.VMEM_SHARED((N, H), _f32),
        pltpu.VMEM_SHARED((NP,), _f32),
        pltpu.SemaphoreType.DMA,
    ],
)


def kernel(x, edge_index, W1, a_src1, a_dst1, b1, W2, a_src2, a_dst2, b2,
           Wout, bout):
    src = edge_index[0].astype(jnp.int32)
    dst = edge_index[1].astype(jnp.int32)
    pad = EP - E
    src2d = jnp.concatenate([src, jnp.zeros((pad,), jnp.int32)]).reshape(-1, K)
    dst2d = jnp.concatenate([dst, jnp.zeros((pad,), jnp.int32)]).reshape(-1, K)

    h1, es1, ed1 = _tc_in(x, W1, a_src1, a_dst1)
    acc1, den1 = _sc_edge(h1, es1.reshape(N), ed1.reshape(N), src2d, dst2d)
    den1 = den1[:, :N].reshape(NC, N, 1)

    h2, es2, ed2 = _tc_mid(acc1, den1, h1, es1, ed1, b1, W2, a_src2, a_dst2)
    acc2, den2 = _sc_edge(h2, es2.reshape(N), ed2.reshape(N), src2d, dst2d)
    den2 = den2[:, :N].reshape(NC, N, 1)

    return _tc_out(acc2, den2, h2, es2, ed2, b2, Wout, bout)


# speedup vs baseline: 14.6632x; 14.6632x over previous
import jax
import jax.numpy as jnp
from jax import lax
from jax.experimental import pallas as pl
from jax.experimental.pallas import tpu as pltpu
from jax.experimental.pallas import tpu_sc as plsc

N = 10000
E = 320000
D = 128
H = 128
O = 128

NC = 2
NS = 16
NW = NC * NS
K = 128
PB = 80
CB = 8
EP = NW * PB * K
NP = 10240
RPT = NP // NS

BLK = 2000
GRID = N // BLK

_f32 = jnp.float32


def _tc_in_body(x_ref, w_ref, asrc_ref, adst_ref, h_ref, es_ref, ed_ref):
    h = jnp.dot(x_ref[...], w_ref[...], preferred_element_type=_f32)
    h_ref[...] = h
    es_ref[...] = jnp.dot(h, asrc_ref[...], preferred_element_type=_f32)
    ed_ref[...] = jnp.dot(h, adst_ref[...], preferred_element_type=_f32)


def _tc_in(x, W, a_src, a_dst):
    return pl.pallas_call(
        _tc_in_body,
        grid=(GRID,),
        in_specs=[
            pl.BlockSpec((BLK, D), lambda i: (i, 0)),
            pl.BlockSpec((D, H), lambda i: (0, 0)),
            pl.BlockSpec((H, 1), lambda i: (0, 0)),
            pl.BlockSpec((H, 1), lambda i: (0, 0)),
        ],
        out_specs=[
            pl.BlockSpec((BLK, H), lambda i: (i, 0)),
            pl.BlockSpec((BLK, 1), lambda i: (i, 0)),
            pl.BlockSpec((BLK, 1), lambda i: (i, 0)),
        ],
        out_shape=[
            jax.ShapeDtypeStruct((N, H), _f32),
            jax.ShapeDtypeStruct((N, 1), _f32),
            jax.ShapeDtypeStruct((N, 1), _f32),
        ],
    )(x, W, a_src.reshape(H, 1), a_dst.reshape(H, 1))


def _combine(acc_ref, den_ref, h_ref, es_ref, ed_ref, b_ref):
    t = es_ref[...] + ed_ref[...]
    sw = jnp.exp(jnp.maximum(t, 0.2 * t))
    num = acc_ref[0] + acc_ref[1] + sw * h_ref[...]
    den = den_ref[0] + den_ref[1] + sw
    return num / den + b_ref[...]


def _tc_mid_body(acc_ref, den_ref, h_ref, es_ref, ed_ref, b_ref, w_ref,
                 asrc_ref, adst_ref, h2_ref, es2_ref, ed2_ref):
    g = jnp.maximum(_combine(acc_ref, den_ref, h_ref, es_ref, ed_ref, b_ref), 0.0)
    h2 = jnp.dot(g, w_ref[...], preferred_element_type=_f32)
    h2_ref[...] = h2
    es2_ref[...] = jnp.dot(h2, asrc_ref[...], preferred_element_type=_f32)
    ed2_ref[...] = jnp.dot(h2, adst_ref[...], preferred_element_type=_f32)


def _tc_mid(acc, den, h, es, ed, b, W, a_src, a_dst):
    return pl.pallas_call(
        _tc_mid_body,
        grid=(GRID,),
        in_specs=[
            pl.BlockSpec((2, BLK, H), lambda i: (0, i, 0)),
            pl.BlockSpec((2, BLK, 1), lambda i: (0, i, 0)),
            pl.BlockSpec((BLK, H), lambda i: (i, 0)),
            pl.BlockSpec((BLK, 1), lambda i: (i, 0)),
            pl.BlockSpec((BLK, 1), lambda i: (i, 0)),
            pl.BlockSpec((1, H), lambda i: (0, 0)),
            pl.BlockSpec((H, H), lambda i: (0, 0)),
            pl.BlockSpec((H, 1), lambda i: (0, 0)),
            pl.BlockSpec((H, 1), lambda i: (0, 0)),
        ],
        out_specs=[
            pl.BlockSpec((BLK, H), lambda i: (i, 0)),
            pl.BlockSpec((BLK, 1), lambda i: (i, 0)),
            pl.BlockSpec((BLK, 1), lambda i: (i, 0)),
        ],
        out_shape=[
            jax.ShapeDtypeStruct((N, H), _f32),
            jax.ShapeDtypeStruct((N, 1), _f32),
            jax.ShapeDtypeStruct((N, 1), _f32),
        ],
    )(acc, den, h, es, ed, b.reshape(1, H), W,
      a_src.reshape(H, 1), a_dst.reshape(H, 1))


def _tc_out_body(acc_ref, den_ref, h_ref, es_ref, ed_ref, b_ref, w_ref,
                 bout_ref, out_ref):
    g = jnp.maximum(_combine(acc_ref, den_ref, h_ref, es_ref, ed_ref, b_ref), 0.0)
    out_ref[...] = (jnp.dot(g, w_ref[...], preferred_element_type=_f32)
                    + bout_ref[...])


def _tc_out(acc, den, h, es, ed, b, Wout, bout):
    return pl.pallas_call(
        _tc_out_body,
        grid=(GRID,),
        in_specs=[
            pl.BlockSpec((2, BLK, H), lambda i: (0, i, 0)),
            pl.BlockSpec((2, BLK, 1), lambda i: (0, i, 0)),
            pl.BlockSpec((BLK, H), lambda i: (i, 0)),
            pl.BlockSpec((BLK, 1), lambda i: (i, 0)),
            pl.BlockSpec((BLK, 1), lambda i: (i, 0)),
            pl.BlockSpec((1, H), lambda i: (0, 0)),
            pl.BlockSpec((H, O), lambda i: (0, 0)),
            pl.BlockSpec((1, O), lambda i: (0, 0)),
        ],
        out_specs=pl.BlockSpec((BLK, O), lambda i: (i, 0)),
        out_shape=jax.ShapeDtypeStruct((N, O), _f32),
    )(acc, den, h, es, ed, b.reshape(1, H), Wout, bout.reshape(1, O))


def _sc_edge_body(h_hbm, es_hbm, ed_hbm, src_hbm, dst_hbm,
                  acc_out, den_out,
                  es_v, ed_v, src_v, dst_v, rows_v, w_v,
                  acc_sh, den_sh, sem):
    c = lax.axis_index("c")
    s = lax.axis_index("s")
    wid = c * NS + s
    base = wid * PB

    pltpu.sync_copy(es_hbm, es_v)
    pltpu.sync_copy(ed_hbm, ed_v)

    zero16 = jnp.zeros((16,), _f32)

    @pl.loop(0, K)
    def _zero_stage(i):
        for j in range(H // 16):
            rows_v[i, pl.ds(j * 16, 16)] = zero16
    for j in range(K // 16):
        w_v[pl.ds(j * 16, 16)] = zero16

    for k in range(RPT // K):
        pltpu.sync_copy(rows_v, acc_sh.at[pl.ds(s * RPT + k * K, K)])
        pltpu.sync_copy(w_v, den_sh.at[pl.ds(s * RPT + k * K, K)])

    plsc.subcore_barrier()

    lane = lax.iota(jnp.int32, 16)

    @pl.loop(0, PB // CB)
    def _chunk(ch):
        pltpu.sync_copy(src_hbm.at[pl.ds(base + ch * CB, CB)], src_v)
        pltpu.sync_copy(dst_hbm.at[pl.ds(base + ch * CB, CB)], dst_v)

        for b in range(CB):
            pltpu.sync_copy(h_hbm.at[src_v.at[b]], rows_v)

            for j in range(K // 16):
                sidx = src_v[b, pl.ds(j * 16, 16)]
                didx = dst_v[b, pl.ds(j * 16, 16)]
                t = (plsc.load_gather(es_v, [sidx])
                     + plsc.load_gather(ed_v, [didx]))
                w = jnp.exp(jnp.maximum(t, 0.2 * t))
                gid = (base + ch * CB + b) * K + j * 16 + lane
                w_v[pl.ds(j * 16, 16)] = jnp.where(gid < E, w, 0.0)

            @pl.loop(0, K)
            def _row(r):
                wr = plsc.load_gather(w_v, [jnp.full((16,), 0, jnp.int32) + r])
                for j in range(H // 16):
                    rows_v[r, pl.ds(j * 16, 16)] = (
                        rows_v[r, pl.ds(j * 16, 16)] * wr)

            pltpu.sync_copy(rows_v, acc_sh.at[dst_v.at[b]], add=True)
            pltpu.sync_copy(w_v, den_sh.at[dst_v.at[b]], add=True)

    plsc.subcore_barrier()

    for k in range(RPT // K):
        off = s * RPT + k * K
        pltpu.sync_copy(acc_sh.at[pl.ds(off, K)], rows_v)
        pltpu.sync_copy(rows_v, acc_out.at[c].at[pl.ds(off, K)])
        pltpu.sync_copy(den_sh.at[pl.ds(off, K)], w_v)
        pltpu.sync_copy(w_v, den_out.at[pl.ds(c * NP + off, K)])


_sc_edge = pl.kernel(
    _sc_edge_body,
    out_type=[
        jax.ShapeDtypeStruct((NC, NP, H), _f32),
        jax.ShapeDtypeStruct((NC * NP,), _f32),
    ],
    mesh=plsc.VectorSubcoreMesh(core_axis_name="c", subcore_axis_name="s",
                                num_cores=NC, num_subcores=NS),
    compiler_params=pltpu.CompilerParams(needs_layout_passes=False),
    scratch_types=[
        pltpu.VMEM((N,), _f32),
        pltpu.VMEM((N,), _f32),
        pltpu.VMEM((CB, K), jnp.int32),
        pltpu.VMEM((CB, K), jnp.int32),
        pltpu.VMEM((K, H), _f32),
        pltpu.VMEM((K,), _f32),
        pltpu.VMEM_SHARED((NP, H), _f32),
        pltpu.VMEM_SHARED((NP,), _f32),
        pltpu.SemaphoreType.DMA,
    ],
)


def kernel(x, edge_index, W1, a_src1, a_dst1, b1, W2, a_src2, a_dst2, b2,
           Wout, bout):
    src = edge_index[0].astype(jnp.int32)
    dst = edge_index[1].astype(jnp.int32)
    pad = EP - E
    src2d = jnp.concatenate([src, jnp.zeros((pad,), jnp.int32)]).reshape(-1, K)
    dst2d = jnp.concatenate([dst, jnp.zeros((pad,), jnp.int32)]).reshape(-1, K)

    h1, es1, ed1 = _tc_in(x, W1, a_src1, a_dst1)
    acc1, den1 = _sc_edge(h1, es1.reshape(N), ed1.reshape(N), src2d, dst2d)
    den1 = den1.reshape(NC, NP, 1)

    h2, es2, ed2 = _tc_mid(acc1, den1, h1, es1, ed1, b1, W2, a_src2, a_dst2)
    acc2, den2 = _sc_edge(h2, es2.reshape(N), ed2.reshape(N), src2d, dst2d)
    den2 = den2.reshape(NC, NP, 1)

    return _tc_out(acc2, den2, h2, es2, ed2, b2, Wout, bout)


# speedup vs baseline: 18.5898x; 1.2678x over previous
import jax
import jax.numpy as jnp
from jax import lax
from jax.experimental import pallas as pl
from jax.experimental.pallas import tpu as pltpu
from jax.experimental.pallas import tpu_sc as plsc

N = 10000
E = 320000
D = 128
H = 128
O = 128

NC = 2
NS = 16
NW = NC * NS
K = 128
PB = 80
CB = 16
EP = NW * PB * K
NP = 10240
RPT = NP // NS

BLK = 2000
GRID = N // BLK

_f32 = jnp.float32


def _tc_in_body(x_ref, w_ref, asrc_ref, adst_ref, h_ref, es_ref, ed_ref):
    h = jnp.dot(x_ref[...], w_ref[...], preferred_element_type=_f32)
    h_ref[...] = h
    es_ref[...] = jnp.dot(h, asrc_ref[...], preferred_element_type=_f32)
    ed_ref[...] = jnp.dot(h, adst_ref[...], preferred_element_type=_f32)


def _tc_in(x, W, a_src, a_dst):
    return pl.pallas_call(
        _tc_in_body,
        grid=(GRID,),
        in_specs=[
            pl.BlockSpec((BLK, D), lambda i: (i, 0)),
            pl.BlockSpec((D, H), lambda i: (0, 0)),
            pl.BlockSpec((H, 1), lambda i: (0, 0)),
            pl.BlockSpec((H, 1), lambda i: (0, 0)),
        ],
        out_specs=[
            pl.BlockSpec((BLK, H), lambda i: (i, 0)),
            pl.BlockSpec((BLK, 1), lambda i: (i, 0)),
            pl.BlockSpec((BLK, 1), lambda i: (i, 0)),
        ],
        out_shape=[
            jax.ShapeDtypeStruct((N, H), _f32),
            jax.ShapeDtypeStruct((N, 1), _f32),
            jax.ShapeDtypeStruct((N, 1), _f32),
        ],
    )(x, W, a_src.reshape(H, 1), a_dst.reshape(H, 1))


def _combine(acc_ref, den_ref, h_ref, es_ref, ed_ref, b_ref):
    t = es_ref[...] + ed_ref[...]
    sw = jnp.exp(jnp.maximum(t, 0.2 * t))
    num = acc_ref[0] + acc_ref[1] + sw * h_ref[...]
    den = den_ref[0] + den_ref[1] + sw
    return num / den + b_ref[...]


def _tc_mid_body(acc_ref, den_ref, h_ref, es_ref, ed_ref, b_ref, w_ref,
                 asrc_ref, adst_ref, h2_ref, es2_ref, ed2_ref):
    g = jnp.maximum(_combine(acc_ref, den_ref, h_ref, es_ref, ed_ref, b_ref), 0.0)
    h2 = jnp.dot(g, w_ref[...], preferred_element_type=_f32)
    h2_ref[...] = h2
    es2_ref[...] = jnp.dot(h2, asrc_ref[...], preferred_element_type=_f32)
    ed2_ref[...] = jnp.dot(h2, adst_ref[...], preferred_element_type=_f32)


def _tc_mid(acc, den, h, es, ed, b, W, a_src, a_dst):
    return pl.pallas_call(
        _tc_mid_body,
        grid=(GRID,),
        in_specs=[
            pl.BlockSpec((2, BLK, H), lambda i: (0, i, 0)),
            pl.BlockSpec((2, BLK, 1), lambda i: (0, i, 0)),
            pl.BlockSpec((BLK, H), lambda i: (i, 0)),
            pl.BlockSpec((BLK, 1), lambda i: (i, 0)),
            pl.BlockSpec((BLK, 1), lambda i: (i, 0)),
            pl.BlockSpec((1, H), lambda i: (0, 0)),
            pl.BlockSpec((H, H), lambda i: (0, 0)),
            pl.BlockSpec((H, 1), lambda i: (0, 0)),
            pl.BlockSpec((H, 1), lambda i: (0, 0)),
        ],
        out_specs=[
            pl.BlockSpec((BLK, H), lambda i: (i, 0)),
            pl.BlockSpec((BLK, 1), lambda i: (i, 0)),
            pl.BlockSpec((BLK, 1), lambda i: (i, 0)),
        ],
        out_shape=[
            jax.ShapeDtypeStruct((N, H), _f32),
            jax.ShapeDtypeStruct((N, 1), _f32),
            jax.ShapeDtypeStruct((N, 1), _f32),
        ],
    )(acc, den, h, es, ed, b.reshape(1, H), W,
      a_src.reshape(H, 1), a_dst.reshape(H, 1))


def _tc_out_body(acc_ref, den_ref, h_ref, es_ref, ed_ref, b_ref, w_ref,
                 bout_ref, out_ref):
    g = jnp.maximum(_combine(acc_ref, den_ref, h_ref, es_ref, ed_ref, b_ref), 0.0)
    out_ref[...] = (jnp.dot(g, w_ref[...], preferred_element_type=_f32)
                    + bout_ref[...])


def _tc_out(acc, den, h, es, ed, b, Wout, bout):
    return pl.pallas_call(
        _tc_out_body,
        grid=(GRID,),
        in_specs=[
            pl.BlockSpec((2, BLK, H), lambda i: (0, i, 0)),
            pl.BlockSpec((2, BLK, 1), lambda i: (0, i, 0)),
            pl.BlockSpec((BLK, H), lambda i: (i, 0)),
            pl.BlockSpec((BLK, 1), lambda i: (i, 0)),
            pl.BlockSpec((BLK, 1), lambda i: (i, 0)),
            pl.BlockSpec((1, H), lambda i: (0, 0)),
            pl.BlockSpec((H, O), lambda i: (0, 0)),
            pl.BlockSpec((1, O), lambda i: (0, 0)),
        ],
        out_specs=pl.BlockSpec((BLK, O), lambda i: (i, 0)),
        out_shape=jax.ShapeDtypeStruct((N, O), _f32),
    )(acc, den, h, es, ed, b.reshape(1, H), Wout, bout.reshape(1, O))


def _sc_edge_body(h_hbm, es_hbm, ed_hbm, src_hbm, dst_hbm,
                  acc_out, den_out,
                  src_v, dst_v, rows_v0, rows_v1, w_v0, w_v1,
                  esg_v0, esg_v1, edg_v0, edg_v1,
                  acc_sh, den_sh, sems):
    c = lax.axis_index("c")
    s = lax.axis_index("s")
    wid = c * NS + s
    base = wid * PB

    rows_v = (rows_v0, rows_v1)
    w_v = (w_v0, w_v1)
    esg_v = (esg_v0, esg_v1)
    edg_v = (edg_v0, edg_v1)

    zero16 = jnp.zeros((16,), _f32)

    @pl.loop(0, K)
    def _zero_stage(i):
        for j in range(H // 16):
            rows_v0[i, pl.ds(j * 16, 16)] = zero16
    for j in range(K // 16):
        w_v0[pl.ds(j * 16, 16)] = zero16

    for k in range(RPT // K):
        pltpu.sync_copy(rows_v0, acc_sh.at[pl.ds(s * RPT + k * K, K)])
        pltpu.sync_copy(w_v0, den_sh.at[pl.ds(s * RPT + k * K, K)])

    plsc.subcore_barrier()

    lane = lax.iota(jnp.int32, 16)

    def _issue_gathers(b):
        i = b & 1
        return (
            pltpu.async_copy(h_hbm.at[src_v.at[b]], rows_v[i], sems.at[i]),
            pltpu.async_copy(es_hbm.at[src_v.at[b]], esg_v[i], sems.at[2 + i]),
            pltpu.async_copy(ed_hbm.at[dst_v.at[b]], edg_v[i], sems.at[4 + i]),
        )

    @pl.loop(0, PB // CB)
    def _chunk(ch):
        pltpu.sync_copy(src_hbm.at[pl.ds(base + ch * CB, CB)], src_v)
        pltpu.sync_copy(dst_hbm.at[pl.ds(base + ch * CB, CB)], dst_v)

        gath = {0: _issue_gathers(0)}
        scat = {}
        for b in range(CB):
            i = b & 1
            if b + 1 < CB:
                if b - 1 >= 0:
                    for d in scat[b - 1]:
                        d.wait()
                gath[b + 1] = _issue_gathers(b + 1)
            for d in gath[b]:
                d.wait()

            for j in range(K // 16):
                t = (esg_v[i][pl.ds(j * 16, 16)]
                     + edg_v[i][pl.ds(j * 16, 16)])
                w = jnp.exp(jnp.maximum(t, 0.2 * t))
                gid = (base + ch * CB + b) * K + j * 16 + lane
                w_v[i][pl.ds(j * 16, 16)] = jnp.where(gid < E, w, 0.0)

            @pl.loop(0, K, unroll=4)
            def _row(r):
                wr = plsc.load_gather(w_v[i], [jnp.full((16,), 0, jnp.int32) + r])
                for j in range(H // 16):
                    rows_v[i][r, pl.ds(j * 16, 16)] = (
                        rows_v[i][r, pl.ds(j * 16, 16)] * wr)

            scat[b] = (
                pltpu.async_copy(rows_v[i], acc_sh.at[dst_v.at[b]],
                                 sems.at[6 + i], add=True),
                pltpu.async_copy(w_v[i], den_sh.at[dst_v.at[b]],
                                 sems.at[8 + i], add=True),
            )
        for b in (CB - 2, CB - 1):
            for d in scat[b]:
                d.wait()

    plsc.subcore_barrier()

    for k in range(RPT // K):
        off = s * RPT + k * K
        pltpu.sync_copy(acc_sh.at[pl.ds(off, K)], rows_v0)
        pltpu.sync_copy(rows_v0, acc_out.at[c].at[pl.ds(off, K)])
        pltpu.sync_copy(den_sh.at[pl.ds(off, K)], w_v0)
        pltpu.sync_copy(w_v0, den_out.at[pl.ds(c * NP + off, K)])


_sc_edge = pl.kernel(
    _sc_edge_body,
    out_type=[
        jax.ShapeDtypeStruct((NC, NP, H), _f32),
        jax.ShapeDtypeStruct((NC * NP,), _f32),
    ],
    mesh=plsc.VectorSubcoreMesh(core_axis_name="c", subcore_axis_name="s",
                                num_cores=NC, num_subcores=NS),
    compiler_params=pltpu.CompilerParams(needs_layout_passes=False),
    scratch_types=[
        pltpu.VMEM((CB, K), jnp.int32),
        pltpu.VMEM((CB, K), jnp.int32),
        pltpu.VMEM((K, H), _f32),
        pltpu.VMEM((K, H), _f32),
        pltpu.VMEM((K,), _f32),
        pltpu.VMEM((K,), _f32),
        pltpu.VMEM((K,), _f32),
        pltpu.VMEM((K,), _f32),
        pltpu.VMEM((K,), _f32),
        pltpu.VMEM((K,), _f32),
        pltpu.VMEM_SHARED((NP, H), _f32),
        pltpu.VMEM_SHARED((NP,), _f32),
        pltpu.SemaphoreType.DMA((10,)),
    ],
)


def kernel(x, edge_index, W1, a_src1, a_dst1, b1, W2, a_src2, a_dst2, b2,
           Wout, bout):
    src = edge_index[0].astype(jnp.int32)
    dst = edge_index[1].astype(jnp.int32)
    pad = EP - E
    src2d = jnp.concatenate([src, jnp.zeros((pad,), jnp.int32)]).reshape(-1, K)
    dst2d = jnp.concatenate([dst, jnp.zeros((pad,), jnp.int32)]).reshape(-1, K)

    h1, es1, ed1 = _tc_in(x, W1, a_src1, a_dst1)
    acc1, den1 = _sc_edge(h1, es1.reshape(N), ed1.reshape(N), src2d, dst2d)
    den1 = den1.reshape(NC, NP, 1)

    h2, es2, ed2 = _tc_mid(acc1, den1, h1, es1, ed1, b1, W2, a_src2, a_dst2)
    acc2, den2 = _sc_edge(h2, es2.reshape(N), ed2.reshape(N), src2d, dst2d)
    den2 = den2.reshape(NC, NP, 1)

    return _tc_out(acc2, den2, h2, es2, ed2, b2, Wout, bout)
